# Initial kernel scaffold; baseline (speedup 1.0000x reference)
import jax
import jax.numpy as jnp
from jax.experimental import pallas as pl


def kernel(query, queue, k):
    raise NotImplementedError("write your pallas kernel here")


# speedup vs baseline: 1.3646x; 1.3646x over previous
import functools

import jax
import jax.numpy as jnp
from jax import lax
from jax.experimental import pallas as pl
from jax.experimental.pallas import tpu as pltpu
from jax.experimental.pallas import tpu_sc as plsc

N = 1024
D = 32
M = 100000
K = 16
CHUNK = 2048
MPAD = 102400
NCHUNK = MPAD // CHUNK
INT_MAX = jnp.iinfo(jnp.int32).max


def _topk_body(q_ref, t_ref, vals_ref, idx_ref):
    i = pl.program_id(0)

    sim = lax.dot_general(
        q_ref[...], t_ref[...], (((1,), (1,)), ((), ())),
        preferred_element_type=jnp.float32)
    gidx = i * CHUNK + lax.broadcasted_iota(jnp.int32, (N, CHUNK), 1)
    sim = jnp.where(gidx < M, sim, -jnp.inf)

    @pl.when(i == 0)
    def _init():
        vals_ref[...] = jnp.full((N, K), -jnp.inf, jnp.float32)
        idx_ref[...] = jnp.full((N, K), INT_MAX, jnp.int32)

    vals = jnp.concatenate([vals_ref[...], sim], axis=1)
    idxs = jnp.concatenate([idx_ref[...], gidx], axis=1)
    ms, ams = [], []
    for _ in range(K):
        m = jnp.max(vals, axis=1, keepdims=True)
        ism = vals == m
        am = jnp.min(jnp.where(ism, idxs, INT_MAX), axis=1, keepdims=True)
        ms.append(m)
        ams.append(am)
        vals = jnp.where(ism & (idxs == am), -jnp.inf, vals)
    vals_ref[...] = jnp.concatenate(ms, axis=1)
    idx_ref[...] = jnp.concatenate(ams, axis=1)


def _topk(query, queue_padded, interpret=False):
    return pl.pallas_call(
        _topk_body,
        grid=(NCHUNK,),
        in_specs=[
            pl.BlockSpec((N, D), lambda i: (0, 0)),
            pl.BlockSpec((CHUNK, D), lambda i: (i, 0)),
        ],
        out_specs=[
            pl.BlockSpec((N, K), lambda i: (0, 0)),
            pl.BlockSpec((N, K), lambda i: (0, 0)),
        ],
        out_shape=[
            jax.ShapeDtypeStruct((N, K), jnp.float32),
            jax.ShapeDtypeStruct((N, K), jnp.int32),
        ],
        compiler_params=pltpu.CompilerParams(
            dimension_semantics=("arbitrary",)),
        interpret=interpret,
    )(query, queue_padded)


_SC_WORKERS = 32
_ROWS_PER_W = (N * K) // _SC_WORKERS
_IDX_CHUNK = 128
_NJ = _ROWS_PER_W // _IDX_CHUNK


def _gather_sc(queue, flat_idx):
    mesh = plsc.VectorSubcoreMesh(core_axis_name="c", subcore_axis_name="s")

    @functools.partial(
        pl.kernel, mesh=mesh,
        out_type=jax.ShapeDtypeStruct((N * K, D), jnp.float32),
        scratch_types=[
            pltpu.VMEM((_IDX_CHUNK,), jnp.int32),
            pltpu.VMEM((_IDX_CHUNK, D), jnp.float32),
            pltpu.SemaphoreType.DMA,
        ],
        compiler_params=pltpu.CompilerParams(use_tc_tiling_on_sc=False),
    )
    def gk(table_hbm, idx_hbm, out_hbm, idx_v, rows_v, sem):
        wid = lax.axis_index("s") * 2 + lax.axis_index("c")
        base = wid * _ROWS_PER_W
        for j in range(_NJ):
            off = base + j * _IDX_CHUNK
            pltpu.sync_copy(idx_hbm.at[pl.ds(off, _IDX_CHUNK)], idx_v)
            pltpu.async_copy(table_hbm.at[idx_v], rows_v, sem).wait()
            pltpu.sync_copy(rows_v, out_hbm.at[pl.ds(off, _IDX_CHUNK)])

    return gk(queue, flat_idx)


def kernel(query, queue, k):
    queue_padded = jnp.pad(queue, ((0, MPAD - M), (0, 0)))
    values, indices = _topk(query, queue_padded)
    neighbors = _gather_sc(queue, indices.reshape(N * K)).reshape(N, K, D)
    values = values + (jnp.asarray(k, jnp.float32) - jnp.float32(K))
    return neighbors, values


# speedup vs baseline: 1.4421x; 1.0568x over previous
import functools

import jax
import jax.numpy as jnp
from jax import lax
from jax.experimental import pallas as pl
from jax.experimental.pallas import tpu as pltpu
from jax.experimental.pallas import tpu_sc as plsc

N = 1024
D = 32
M = 100000
K = 16
CHUNK = 2048
MPAD = 102400
NCHUNK = MPAD // CHUNK
INT_MAX = jnp.iinfo(jnp.int32).max


def _topk_body(q_ref, t_ref, vals_ref, idx_ref):
    i = pl.program_id(0)

    sim = lax.dot_general(
        q_ref[...], t_ref[...], (((1,), (1,)), ((), ())),
        preferred_element_type=jnp.float32)
    lidx = lax.broadcasted_iota(jnp.int32, (N, CHUNK), 1)
    sim = jnp.where(lidx >= M - i * CHUNK, -jnp.inf, sim)

    ms, ams = [], []
    s = sim
    for j in range(K):
        m = jnp.max(s, axis=1, keepdims=True)
        am = jnp.min(jnp.where(s == m, lidx, INT_MAX), axis=1, keepdims=True)
        ms.append(m)
        ams.append(am)
        if j < K - 1:
            s = jnp.where(lidx == am, -jnp.inf, s)
    cvals = jnp.concatenate(ms, axis=1)
    cidx = i * CHUNK + jnp.concatenate(ams, axis=1)

    @pl.when(i == 0)
    def _init():
        vals_ref[...] = jnp.full((N, K), -jnp.inf, jnp.float32)
        idx_ref[...] = jnp.full((N, K), INT_MAX, jnp.int32)

    mv = jnp.concatenate([vals_ref[...], cvals], axis=1)
    mi = jnp.concatenate([idx_ref[...], cidx], axis=1)
    oms, oams = [], []
    for j in range(K):
        m = jnp.max(mv, axis=1, keepdims=True)
        sel = jnp.min(jnp.where(mv == m, mi, INT_MAX), axis=1, keepdims=True)
        oms.append(m)
        oams.append(sel)
        if j < K - 1:
            mv = jnp.where(mi == sel, -jnp.inf, mv)
    vals_ref[...] = jnp.concatenate(oms, axis=1)
    idx_ref[...] = jnp.concatenate(oams, axis=1)


def _topk(query, queue_padded, interpret=False):
    return pl.pallas_call(
        _topk_body,
        grid=(NCHUNK,),
        in_specs=[
            pl.BlockSpec((N, D), lambda i: (0, 0)),
            pl.BlockSpec((CHUNK, D), lambda i: (i, 0)),
        ],
        out_specs=[
            pl.BlockSpec((N, K), lambda i: (0, 0)),
            pl.BlockSpec((N, K), lambda i: (0, 0)),
        ],
        out_shape=[
            jax.ShapeDtypeStruct((N, K), jnp.float32),
            jax.ShapeDtypeStruct((N, K), jnp.int32),
        ],
        compiler_params=pltpu.CompilerParams(
            dimension_semantics=("arbitrary",)),
        interpret=interpret,
    )(query, queue_padded)


_SC_WORKERS = 32
_ROWS_PER_W = (N * K) // _SC_WORKERS
_IDX_CHUNK = 128
_NJ = _ROWS_PER_W // _IDX_CHUNK


def _gather_sc(queue, flat_idx):
    mesh = plsc.VectorSubcoreMesh(core_axis_name="c", subcore_axis_name="s")

    @functools.partial(
        pl.kernel, mesh=mesh,
        out_type=jax.ShapeDtypeStruct((N * K, D), jnp.float32),
        scratch_types=[
            pltpu.VMEM((_IDX_CHUNK,), jnp.int32),
            pltpu.VMEM((_IDX_CHUNK, D), jnp.float32),
            pltpu.SemaphoreType.DMA,
        ],
        compiler_params=pltpu.CompilerParams(use_tc_tiling_on_sc=False),
    )
    def gk(table_hbm, idx_hbm, out_hbm, idx_v, rows_v, sem):
        wid = lax.axis_index("s") * 2 + lax.axis_index("c")
        base = wid * _ROWS_PER_W
        for j in range(_NJ):
            off = base + j * _IDX_CHUNK
            pltpu.sync_copy(idx_hbm.at[pl.ds(off, _IDX_CHUNK)], idx_v)
            pltpu.async_copy(table_hbm.at[idx_v], rows_v, sem).wait()
            pltpu.sync_copy(rows_v, out_hbm.at[pl.ds(off, _IDX_CHUNK)])

    return gk(queue, flat_idx)


def kernel(query, queue, k):
    queue_padded = jnp.pad(queue, ((0, MPAD - M), (0, 0)))
    values, indices = _topk(query, queue_padded)
    neighbors = _gather_sc(queue, indices.reshape(N * K)).reshape(N, K, D)
    values = values + (jnp.asarray(k, jnp.float32) - jnp.float32(K))
    return neighbors, values


# speedup vs baseline: 2.7557x; 1.9109x over previous
import functools

import jax
import jax.numpy as jnp
from jax import lax
from jax.experimental import pallas as pl
from jax.experimental.pallas import tpu as pltpu
from jax.experimental.pallas import tpu_sc as plsc

N = 1024
D = 32
M = 100000
K = 16
CHUNK = 2048
MPAD = 102400
NCHUNK = MPAD // CHUNK
INT_MAX = jnp.iinfo(jnp.int32).max


def _topk_body(q_ref, t_ref, vals_ref, idx_ref):
    i = pl.program_id(0)

    sim = lax.dot_general(
        q_ref[...], t_ref[...], (((1,), (1,)), ((), ())),
        preferred_element_type=jnp.float32)
    lidx = lax.broadcasted_iota(jnp.int32, (N, CHUNK), 1)
    sim = jnp.where(lidx >= M - i * CHUNK, -jnp.inf, sim)

    @pl.when(i == 0)
    def _bootstrap():
        ms, ams = [], []
        s = sim
        for j in range(K):
            m = jnp.max(s, axis=1, keepdims=True)
            am = jnp.min(jnp.where(s == m, lidx, INT_MAX), axis=1,
                         keepdims=True)
            ms.append(m)
            ams.append(am)
            if j < K - 1:
                s = jnp.where(lidx == am, -jnp.inf, s)
        vals_ref[...] = jnp.concatenate(ms, axis=1)
        idx_ref[...] = jnp.concatenate(ams, axis=1)

    @pl.when(i > 0)
    def _threshold_extract():
        rv0 = vals_ref[...]
        ri0 = idx_ref[...]
        m0 = jnp.max(sim, axis=1, keepdims=True)

        def cond(carry):
            _, rv, _, m = carry
            return jnp.any(m > rv[:, K - 1:K])

        def body(carry):
            s, rv, ri, m = carry
            am = jnp.min(jnp.where(s == m, lidx, INT_MAX), axis=1,
                         keepdims=True)
            gi = am + i * CHUNK
            ge = (rv > m) | ((rv == m) & (ri < gi))
            ge_i = ge.astype(jnp.int32)
            ge_s = jnp.concatenate(
                [jnp.ones((N, 1), jnp.int32), ge_i[:, :K - 1]], axis=1) != 0
            rv_s = jnp.concatenate(
                [jnp.full((N, 1), -jnp.inf, jnp.float32), rv[:, :K - 1]],
                axis=1)
            ri_s = jnp.concatenate(
                [jnp.full((N, 1), INT_MAX, jnp.int32), ri[:, :K - 1]],
                axis=1)
            mb = jnp.broadcast_to(m, (N, K))
            gib = jnp.broadcast_to(gi, (N, K))
            rv = jnp.where(ge, rv, jnp.where(ge_s, mb, rv_s))
            ri = jnp.where(ge, ri, jnp.where(ge_s, gib, ri_s))
            s = jnp.where(lidx == am, -jnp.inf, s)
            m = jnp.max(s, axis=1, keepdims=True)
            return s, rv, ri, m

        _, rv, ri, _ = lax.while_loop(cond, body, (sim, rv0, ri0, m0))
        vals_ref[...] = rv
        idx_ref[...] = ri


def _topk(query, queue_padded, interpret=False):
    return pl.pallas_call(
        _topk_body,
        grid=(NCHUNK,),
        in_specs=[
            pl.BlockSpec((N, D), lambda i: (0, 0)),
            pl.BlockSpec((CHUNK, D), lambda i: (i, 0)),
        ],
        out_specs=[
            pl.BlockSpec((N, K), lambda i: (0, 0)),
            pl.BlockSpec((N, K), lambda i: (0, 0)),
        ],
        out_shape=[
            jax.ShapeDtypeStruct((N, K), jnp.float32),
            jax.ShapeDtypeStruct((N, K), jnp.int32),
        ],
        compiler_params=pltpu.CompilerParams(
            dimension_semantics=("arbitrary",)),
        interpret=interpret,
    )(query, queue_padded)


_SC_WORKERS = 32
_ROWS_PER_W = (N * K) // _SC_WORKERS
_IDX_CHUNK = 128
_NJ = _ROWS_PER_W // _IDX_CHUNK


def _gather_sc(queue, flat_idx):
    mesh = plsc.VectorSubcoreMesh(core_axis_name="c", subcore_axis_name="s")

    @functools.partial(
        pl.kernel, mesh=mesh,
        out_type=jax.ShapeDtypeStruct((N * K, D), jnp.float32),
        scratch_types=[
            pltpu.VMEM((_IDX_CHUNK,), jnp.int32),
            pltpu.VMEM((_IDX_CHUNK, D), jnp.float32),
            pltpu.SemaphoreType.DMA,
        ],
        compiler_params=pltpu.CompilerParams(use_tc_tiling_on_sc=False),
    )
    def gk(table_hbm, idx_hbm, out_hbm, idx_v, rows_v, sem):
        wid = lax.axis_index("s") * 2 + lax.axis_index("c")
        base = wid * _ROWS_PER_W
        for j in range(_NJ):
            off = base + j * _IDX_CHUNK
            pltpu.sync_copy(idx_hbm.at[pl.ds(off, _IDX_CHUNK)], idx_v)
            pltpu.async_copy(table_hbm.at[idx_v], rows_v, sem).wait()
            pltpu.sync_copy(rows_v, out_hbm.at[pl.ds(off, _IDX_CHUNK)])

    return gk(queue, flat_idx)


def kernel(query, queue, k):
    queue_padded = jnp.pad(queue, ((0, MPAD - M), (0, 0)))
    values, indices = _topk(query, queue_padded)
    neighbors = _gather_sc(queue, indices.reshape(N * K)).reshape(N, K, D)
    values = values + (jnp.asarray(k, jnp.float32) - jnp.float32(K))
    return neighbors, values


# speedup vs baseline: 2.9787x; 1.0809x over previous
import functools

import jax
import jax.numpy as jnp
from jax import lax
from jax.experimental import pallas as pl
from jax.experimental.pallas import tpu as pltpu
from jax.experimental.pallas import tpu_sc as plsc

N = 1024
D = 32
M = 100000
K = 16
CHUNK = 1024
NCHUNK = -(-M // CHUNK)
MPAD = CHUNK * NCHUNK
INT_MAX = jnp.iinfo(jnp.int32).max


def _topk_body(q_ref, t_ref, vals_ref, idx_ref):
    i = pl.program_id(0)

    sim = lax.dot_general(
        q_ref[...], t_ref[...], (((1,), (1,)), ((), ())),
        preferred_element_type=jnp.float32)
    lidx = lax.broadcasted_iota(jnp.int32, (N, CHUNK), 1)
    sim = jnp.where(lidx >= M - i * CHUNK, -jnp.inf, sim)

    @pl.when(i == 0)
    def _bootstrap():
        ms, ams = [], []
        s = sim
        for j in range(K):
            m = jnp.max(s, axis=1, keepdims=True)
            am = jnp.min(jnp.where(s == m, lidx, INT_MAX), axis=1,
                         keepdims=True)
            ms.append(m)
            ams.append(am)
            if j < K - 1:
                s = jnp.where(lidx == am, -jnp.inf, s)
        vals_ref[...] = jnp.concatenate(ms, axis=1)
        idx_ref[...] = jnp.concatenate(ams, axis=1)

    @pl.when(i > 0)
    def _threshold_extract():
        rv0 = vals_ref[...]
        ri0 = idx_ref[...]
        m0 = jnp.max(sim, axis=1, keepdims=True)

        def cond(carry):
            _, rv, _, m = carry
            return jnp.any(m > rv[:, K - 1:K])

        def body(carry):
            s, rv, ri, m = carry
            am = jnp.min(jnp.where(s == m, lidx, INT_MAX), axis=1,
                         keepdims=True)
            gi = am + i * CHUNK
            ge = (rv > m) | ((rv == m) & (ri < gi))
            ge_i = ge.astype(jnp.int32)
            ge_s = jnp.concatenate(
                [jnp.ones((N, 1), jnp.int32), ge_i[:, :K - 1]], axis=1) != 0
            rv_s = jnp.concatenate(
                [jnp.full((N, 1), -jnp.inf, jnp.float32), rv[:, :K - 1]],
                axis=1)
            ri_s = jnp.concatenate(
                [jnp.full((N, 1), INT_MAX, jnp.int32), ri[:, :K - 1]],
                axis=1)
            mb = jnp.broadcast_to(m, (N, K))
            gib = jnp.broadcast_to(gi, (N, K))
            rv = jnp.where(ge, rv, jnp.where(ge_s, mb, rv_s))
            ri = jnp.where(ge, ri, jnp.where(ge_s, gib, ri_s))
            s = jnp.where(lidx == am, -jnp.inf, s)
            m = jnp.max(s, axis=1, keepdims=True)
            return s, rv, ri, m

        _, rv, ri, _ = lax.while_loop(cond, body, (sim, rv0, ri0, m0))
        vals_ref[...] = rv
        idx_ref[...] = ri


def _topk(query, queue_padded, interpret=False):
    return pl.pallas_call(
        _topk_body,
        grid=(NCHUNK,),
        in_specs=[
            pl.BlockSpec((N, D), lambda i: (0, 0)),
            pl.BlockSpec((CHUNK, D), lambda i: (i, 0)),
        ],
        out_specs=[
            pl.BlockSpec((N, K), lambda i: (0, 0)),
            pl.BlockSpec((N, K), lambda i: (0, 0)),
        ],
        out_shape=[
            jax.ShapeDtypeStruct((N, K), jnp.float32),
            jax.ShapeDtypeStruct((N, K), jnp.int32),
        ],
        compiler_params=pltpu.CompilerParams(
            dimension_semantics=("arbitrary",)),
        interpret=interpret,
    )(query, queue_padded)


_SC_WORKERS = 32
_ROWS_PER_W = (N * K) // _SC_WORKERS
_IDX_CHUNK = 128
_NJ = _ROWS_PER_W // _IDX_CHUNK


def _gather_sc(queue, flat_idx):
    mesh = plsc.VectorSubcoreMesh(core_axis_name="c", subcore_axis_name="s")

    @functools.partial(
        pl.kernel, mesh=mesh,
        out_type=jax.ShapeDtypeStruct((N * K, D), jnp.float32),
        scratch_types=[
            pltpu.VMEM((_IDX_CHUNK,), jnp.int32),
            pltpu.VMEM((_IDX_CHUNK, D), jnp.float32),
            pltpu.SemaphoreType.DMA,
        ],
        compiler_params=pltpu.CompilerParams(use_tc_tiling_on_sc=False),
    )
    def gk(table_hbm, idx_hbm, out_hbm, idx_v, rows_v, sem):
        wid = lax.axis_index("s") * 2 + lax.axis_index("c")
        base = wid * _ROWS_PER_W
        for j in range(_NJ):
            off = base + j * _IDX_CHUNK
            pltpu.sync_copy(idx_hbm.at[pl.ds(off, _IDX_CHUNK)], idx_v)
            pltpu.async_copy(table_hbm.at[idx_v], rows_v, sem).wait()
            pltpu.sync_copy(rows_v, out_hbm.at[pl.ds(off, _IDX_CHUNK)])

    return gk(queue, flat_idx)


def kernel(query, queue, k):
    queue_padded = jnp.pad(queue, ((0, MPAD - M), (0, 0)))
    values, indices = _topk(query, queue_padded)
    neighbors = _gather_sc(queue, indices.reshape(N * K)).reshape(N, K, D)
    values = values + (jnp.asarray(k, jnp.float32) - jnp.float32(K))
    return neighbors, values
